# baseline (speedup 1.0000x reference)
import jax
import jax.numpy as jnp
from jax.experimental import pallas as pl
from jax.experimental.pallas import tpu as pltpu

_BM = 200
_NSLOTS = 4


def _sgc_body(w_ref, b_ref, x_hbm, adj_hbm, o_ref,
              xw_ref, x_ref, buf, sems, x_sem):
    n = o_ref.shape[0]
    nb = n // _BM

    def copy(blk, slot):
        return pltpu.make_async_copy(
            adj_hbm.at[pl.ds(blk * _BM, _BM), :],
            buf.at[slot],
            sems.at[slot],
        )

    x_copy = pltpu.make_async_copy(x_hbm, x_ref, x_sem)
    x_copy.start()
    for s in range(_NSLOTS):
        copy(s, s).start()

    x_copy.wait()
    xw_ref[...] = jax.lax.dot_general(
        x_ref[...], w_ref[...],
        (((1,), (1,)), ((), ())),
        preferred_element_type=jnp.float32,
    )

    bias = b_ref[...]
    for blk in range(nb):
        slot = blk % _NSLOTS
        copy(blk, slot).wait()
        o_ref[blk * _BM:(blk + 1) * _BM, :] = (
            jnp.dot(buf[slot], xw_ref[...], preferred_element_type=jnp.float32)
            + bias
        )
        nxt = blk + _NSLOTS
        if nxt < nb:
            copy(nxt, slot).start()


def kernel(x, adj, W, b):
    n, nfeat = x.shape
    nclass = W.shape[0]
    b2 = b.reshape(1, nclass)
    out = pl.pallas_call(
        _sgc_body,
        in_specs=[
            pl.BlockSpec(memory_space=pltpu.MemorySpace.VMEM),
            pl.BlockSpec(memory_space=pltpu.MemorySpace.VMEM),
            pl.BlockSpec(memory_space=pltpu.MemorySpace.HBM),
            pl.BlockSpec(memory_space=pltpu.MemorySpace.HBM),
        ],
        out_specs=pl.BlockSpec(memory_space=pltpu.MemorySpace.VMEM),
        out_shape=jax.ShapeDtypeStruct((n, nclass), jnp.float32),
        scratch_shapes=[
            pltpu.VMEM((n, nclass), jnp.float32),
            pltpu.VMEM((n, nfeat), jnp.float32),
            pltpu.VMEM((_NSLOTS, _BM, n), jnp.float32),
            pltpu.SemaphoreType.DMA((_NSLOTS,)),
            pltpu.SemaphoreType.DMA,
        ],
    )(W, b2, x, adj)
    return out


# speedup vs baseline: 1.0200x; 1.0200x over previous
import jax
import jax.numpy as jnp
from jax.experimental import pallas as pl
from jax.experimental.pallas import tpu as pltpu

_BM = 200
_NS = 5


def _sgc_body(w_ref, b_ref, x_hbm, adj_hbm, o_hbm,
              xw_ref, x_ref, ostage, buf, sems, x_sem, o_sem):
    n = x_ref.shape[0]
    nb = n // _BM

    def adj_copy(blk):
        return pltpu.make_async_copy(
            adj_hbm.at[pl.ds(blk * _BM, _BM), :],
            buf.at[blk % _NS],
            sems.at[blk % _NS],
        )

    x_copy = pltpu.make_async_copy(x_hbm, x_ref, x_sem)
    x_copy.start()
    for j in range(_NS - 1):
        adj_copy(j).start()

    x_copy.wait()
    xw_ref[...] = jax.lax.dot_general(
        x_ref[...], w_ref[...],
        (((1,), (1,)), ((), ())),
        preferred_element_type=jnp.float32,
    )
    bias = b_ref[...]

    out_copies = []
    for blk in range(nb):
        adj_copy(blk).wait()
        nxt = blk + _NS - 1
        if nxt < nb:
            adj_copy(nxt).start()
        ostage[blk * _BM:(blk + 1) * _BM, :] = (
            jnp.dot(buf[blk % _NS], xw_ref[...],
                    preferred_element_type=jnp.float32)
            + bias
        )
        oc = pltpu.make_async_copy(
            ostage.at[pl.ds(blk * _BM, _BM), :],
            o_hbm.at[pl.ds(blk * _BM, _BM), :],
            o_sem,
        )
        oc.start()
        out_copies.append(oc)

    for oc in out_copies:
        oc.wait()


def kernel(x, adj, W, b):
    n, nfeat = x.shape
    nclass = W.shape[0]
    b2 = b.reshape(1, nclass)
    out = pl.pallas_call(
        _sgc_body,
        in_specs=[
            pl.BlockSpec(memory_space=pltpu.MemorySpace.VMEM),
            pl.BlockSpec(memory_space=pltpu.MemorySpace.VMEM),
            pl.BlockSpec(memory_space=pltpu.MemorySpace.HBM),
            pl.BlockSpec(memory_space=pltpu.MemorySpace.HBM),
        ],
        out_specs=pl.BlockSpec(memory_space=pltpu.MemorySpace.HBM),
        out_shape=jax.ShapeDtypeStruct((n, nclass), jnp.float32),
        scratch_shapes=[
            pltpu.VMEM((n, nclass), jnp.float32),
            pltpu.VMEM((n, nfeat), jnp.float32),
            pltpu.VMEM((n, nclass), jnp.float32),
            pltpu.VMEM((_NS, _BM, n), jnp.float32),
            pltpu.SemaphoreType.DMA((_NS,)),
            pltpu.SemaphoreType.DMA,
            pltpu.SemaphoreType.DMA,
        ],
    )(W, b2, x, adj)
    return out
